# baseline (speedup 1.0000x reference)
import functools

import jax
import jax.numpy as jnp
from jax import lax
from jax.experimental import pallas as pl
from jax.experimental.pallas import tpu as pltpu
from jax.experimental.pallas import tpu_sc as plsc

F32 = jnp.float32
NC = 2
NS = 16
NW = NC * NS
NACC = 5120


def _mesh():
    return plsc.VectorSubcoreMesh(core_axis_name="c", subcore_axis_name="s")


def _col_groups(ch):
    offs = list(range(0, ch - 15, 16))
    if ch % 16:
        offs.append(ch - 16)
    return offs


def _sc_degree(dst2, n_pad):
    rt, ch = dst2.shape
    rows_per_tile = rt // NW
    seg = n_pad // NS

    @functools.partial(
        pl.kernel,
        out_type=jax.ShapeDtypeStruct((NC * n_pad,), F32),
        mesh=_mesh(),
        scratch_types=[
            pltpu.VMEM((rows_per_tile, ch), jnp.int32),
            pltpu.VMEM((ch,), F32),
            pltpu.VMEM((seg,), F32),
            pltpu.VMEM_SHARED((n_pad,), F32),
        ],
    )
    def deg_kernel(dst_hbm, out_hbm, idx_v, ones_v, zbuf_v, acc_sh):
        cid = lax.axis_index("c")
        sid = lax.axis_index("s")
        wid = sid * NC + cid

        def fill_zero(i, carry):
            zbuf_v[pl.ds(i * 16, 16)] = jnp.zeros((16,), F32)
            return carry

        lax.fori_loop(0, seg // 16, fill_zero, 0)
        for k in _col_groups(ch):
            ones_v[pl.ds(k, 16)] = jnp.ones((16,), F32)

        pltpu.sync_copy(zbuf_v, acc_sh.at[pl.ds(sid * seg, seg)])
        plsc.subcore_barrier()

        pltpu.sync_copy(dst_hbm.at[pl.ds(wid * rows_per_tile, rows_per_tile)],
                        idx_v)

        def body(j, carry):
            pltpu.sync_copy(ones_v, acc_sh.at[idx_v.at[j]], add=True)
            return carry

        lax.fori_loop(0, rows_per_tile, body, 0)

        plsc.subcore_barrier()
        pltpu.sync_copy(acc_sh.at[pl.ds(sid * seg, seg)],
                        out_hbm.at[pl.ds(cid * n_pad + sid * seg, seg)])

    return deg_kernel(dst2)


def _sc_agg(table2, src1, dst1, ch):
    e = src1.shape[0]
    ept = e // NW
    nchunk = ept // ch
    seg = NACC // NS
    cpy = 64

    @functools.partial(
        pl.kernel,
        out_type=jax.ShapeDtypeStruct((NC, NACC, 128), F32),
        mesh=_mesh(),
        scratch_types=[
            pltpu.VMEM((ept,), jnp.int32),
            pltpu.VMEM((ept,), jnp.int32),
            pltpu.VMEM((nchunk, ch), jnp.int32),
            pltpu.VMEM((ch, 128), F32),
            pltpu.VMEM((ch, 128), F32),
            pltpu.VMEM((cpy, 128), F32),
            pltpu.VMEM_SHARED((NACC, 128), F32),
            pltpu.SemaphoreType.DMA,
            pltpu.SemaphoreType.DMA,
        ],
    )
    def agg_kernel(t2_hbm, src_hbm, dst_hbm, out_hbm,
                   si_v, di_v, ai_v, rows0_v, rows1_v, zbuf_v, acc_sh,
                   sem0, sem1):
        cid = lax.axis_index("c")
        sid = lax.axis_index("s")
        wid = sid * NC + cid

        def zero_row(i, carry):
            for k in range(8):
                zbuf_v[i, pl.ds(k * 16, 16)] = jnp.zeros((16,), F32)
            return carry

        lax.fori_loop(0, cpy, zero_row, 0)
        for k in range(seg // cpy):
            pltpu.sync_copy(zbuf_v,
                            acc_sh.at[pl.ds(sid * seg + k * cpy, cpy)])
        plsc.subcore_barrier()

        pltpu.sync_copy(src_hbm.at[pl.ds(wid * ept, ept)], si_v)
        pltpu.sync_copy(dst_hbm.at[pl.ds(wid * ept, ept)], di_v)

        def xform_g(i, carry):
            s = si_v[pl.ds(i * 16, 16)]
            d = di_v[pl.ds(i * 16, 16)]
            si_v[pl.ds(i * 16, 16)] = s * 2 + (d & 1)
            return carry

        lax.fori_loop(0, ept // 16, xform_g, 0)

        groups = _col_groups(ch)

        def xform_a(r, carry):
            for co in groups:
                d = di_v[pl.ds(r * ch + co, 16)]
                ai_v[r, pl.ds(co, 16)] = d >> 1
            return carry

        lax.fori_loop(0, nchunk, xform_a, 0)

        def gstart(j, buf, sem):
            return pltpu.async_copy(
                t2_hbm.at[si_v.at[pl.ds(j * ch, ch)]], buf, sem)

        def gwait(j, buf, sem):
            pltpu.make_async_copy(
                t2_hbm.at[si_v.at[pl.ds(j * ch, ch)]], buf, sem).wait()

        def sstart(j, buf, sem):
            pltpu.async_copy(buf, acc_sh.at[ai_v.at[j]], sem, add=True)

        def swait(j, buf, sem):
            pltpu.make_async_copy(buf, acc_sh.at[ai_v.at[j]], sem).wait()

        gstart(0, rows0_v, sem0)

        def body2(jj, carry):
            j0 = jj * 2

            @pl.when(jj > 0)
            def _():
                swait(j0 - 1, rows1_v, sem1)

            gstart(j0 + 1, rows1_v, sem1)
            gwait(j0, rows0_v, sem0)
            sstart(j0, rows0_v, sem0)
            gwait(j0 + 1, rows1_v, sem1)
            sstart(j0 + 1, rows1_v, sem1)
            swait(j0, rows0_v, sem0)

            @pl.when(jj + 1 < nchunk // 2)
            def _():
                gstart(j0 + 2, rows0_v, sem0)

            return carry

        lax.fori_loop(0, nchunk // 2, body2, 0)
        swait(nchunk - 1, rows1_v, sem1)

        plsc.subcore_barrier()
        for k in range(seg // cpy):
            pltpu.sync_copy(acc_sh.at[pl.ds(sid * seg + k * cpy, cpy)],
                            out_hbm.at[cid, pl.ds(sid * seg + k * cpy, cpy)])

    return agg_kernel(table2, src1, dst1)


def _sc_edge_hadamard(zpad, ei2, ej2, f_out):
    n, f = zpad.shape
    rt, ch = ei2.shape
    rows_per_tile = rt // NW
    bsz = rt * ch

    @functools.partial(
        pl.kernel,
        out_type=jax.ShapeDtypeStruct((bsz, f_out), F32),
        mesh=_mesh(),
        scratch_types=[
            pltpu.VMEM((rows_per_tile, ch), jnp.int32),
            pltpu.VMEM((rows_per_tile, ch), jnp.int32),
            pltpu.VMEM((ch, f), F32),
            pltpu.VMEM((ch, f), F32),
            pltpu.VMEM((ch, f), F32),
            pltpu.VMEM((ch, f), F32),
            pltpu.VMEM((ch, f_out), F32),
            pltpu.VMEM((ch, f_out), F32),
            pltpu.SemaphoreType.DMA,
            pltpu.SemaphoreType.DMA,
        ],
    )
    def gat_kernel(z_hbm, ei_hbm, ej_hbm, prod_hbm,
                   ii_v, jj_v, bi0_v, bj0_v, bi1_v, bj1_v, pb0_v, pb1_v,
                   semA, semB):
        cid = lax.axis_index("c")
        sid = lax.axis_index("s")
        wid = sid * NC + cid
        base = wid * rows_per_tile * ch

        pltpu.sync_copy(ei_hbm.at[pl.ds(wid * rows_per_tile, rows_per_tile)],
                        ii_v)
        pltpu.sync_copy(ej_hbm.at[pl.ds(wid * rows_per_tile, rows_per_tile)],
                        jj_v)

        def gstart(j, bi, bj, sem):
            pltpu.async_copy(z_hbm.at[ii_v.at[j]], bi, sem)
            pltpu.async_copy(z_hbm.at[jj_v.at[j]], bj, sem)

        def gwait(j, bi, bj, sem):
            pltpu.make_async_copy(z_hbm.at[ii_v.at[j]], bi, sem).wait()
            pltpu.make_async_copy(z_hbm.at[jj_v.at[j]], bj, sem).wait()

        def consume(j, bi, bj, pb):
            def had(rr, c):
                for u in range(4):
                    r = rr * 4 + u
                    for k in range(f_out // 16):
                        a = bi[r, pl.ds(k * 16, 16)]
                        b = bj[r, pl.ds(k * 16, 16)]
                        pb[r, pl.ds(k * 16, 16)] = a * b
                return c

            lax.fori_loop(0, ch // 4, had, 0)
            pltpu.sync_copy(pb, prod_hbm.at[pl.ds(base + j * ch, ch)])

        gstart(0, bi0_v, bj0_v, semA)

        def body2(jj, carry):
            j0 = jj * 2
            gstart(j0 + 1, bi1_v, bj1_v, semB)
            gwait(j0, bi0_v, bj0_v, semA)
            consume(j0, bi0_v, bj0_v, pb0_v)

            @pl.when(jj + 1 < rows_per_tile // 2)
            def _():
                gstart(j0 + 2, bi0_v, bj0_v, semA)

            gwait(j0 + 1, bi1_v, bj1_v, semB)
            consume(j0 + 1, bi1_v, bj1_v, pb1_v)
            return carry

        lax.fori_loop(0, rows_per_tile // 2, body2, 0)

    return gat_kernel(zpad, ei2, ej2)


def _pack_pairs(hh, n, h):
    zpad = jnp.zeros((n, 128 - h), F32)
    row0 = jnp.concatenate([hh, zpad], axis=1)
    row1 = jnp.concatenate([zpad, hh], axis=1)
    return jnp.stack([row0, row1], axis=1)


def _tc_prep(features, w1, degpart3):
    n, d = features.shape
    h = w1.shape[1]
    npad = degpart3.shape[1]

    def mm_body(x_ref, w_ref, h_ref):
        h_ref[...] = jnp.dot(x_ref[...], w_ref[...],
                             preferred_element_type=F32)

    h1 = pl.pallas_call(
        mm_body,
        out_shape=jax.ShapeDtypeStruct((n, h), F32),
        compiler_params=pltpu.CompilerParams(vmem_limit_bytes=100 * 2**20),
    )(features, w1)

    def body(h_ref, deg_ref, dinv_ref, t3_ref):
        deg = deg_ref[...]
        degsum = deg[0] + deg[1] + 1.0
        dinv = lax.rsqrt(degsum[:n])
        dinv_ref[...] = dinv
        hh = h_ref[...] * dinv
        t3_ref[...] = _pack_pairs(hh, n, h)

    return pl.pallas_call(
        body,
        out_shape=[jax.ShapeDtypeStruct((n, 1), F32),
                   jax.ShapeDtypeStruct((n, 2, 128), F32)],
        compiler_params=pltpu.CompilerParams(vmem_limit_bytes=100 * 2**20),
    )(h1, degpart3)


def _tc_combine(aggr, t3, dinv, b, g, be, w_next):
    n = t3.shape[0]
    f = b.shape[1]
    h2 = w_next.shape[1]

    def body(agg_ref, t3_ref, dinv_ref, b_ref, g_ref, be_ref, wn_ref,
             out_ref, t3n_ref):
        agg = agg_ref[...]
        dinv = dinv_ref[...]
        hh = t3_ref[:, 0, :f]
        pre = (agg[0, :n] + agg[1, :n] + hh) * dinv + b_ref[...]
        m = jnp.mean(pre, axis=0, keepdims=True)
        xc = pre - m
        v = jnp.mean(xc * xc, axis=0, keepdims=True)
        xn = xc * lax.rsqrt(v + 1e-5) * g_ref[...] + be_ref[...]
        out = jnp.where(xn > 0, xn, jnp.exp(xn) - 1.0)
        out_ref[...] = out
        hhn = jnp.dot(out, wn_ref[...], preferred_element_type=F32) * dinv
        t3n_ref[...] = _pack_pairs(hhn, n, h2)

    return pl.pallas_call(
        body,
        out_shape=[jax.ShapeDtypeStruct((n, f), F32),
                   jax.ShapeDtypeStruct((n, 2, 128), F32)],
        compiler_params=pltpu.CompilerParams(vmem_limit_bytes=100 * 2**20),
    )(aggr, t3, dinv, b, g, be, w_next)


def _tc_final(aggr, t3, dinv, b, g, be, out1, out2, jk3):
    n = t3.shape[0]
    f = b.shape[1]

    def body(agg_ref, t3_ref, dinv_ref, b_ref, g_ref, be_ref,
             o1_ref, o2_ref, jk_ref, z_ref, zp_ref):
        agg = agg_ref[...]
        hh = t3_ref[:, 0, :f]
        pre = (agg[0, :n] + agg[1, :n] + hh) * dinv_ref[...] + b_ref[...]
        m = jnp.mean(pre, axis=0, keepdims=True)
        xc = pre - m
        v = jnp.mean(xc * xc, axis=0, keepdims=True)
        xn = xc * lax.rsqrt(v + 1e-5) * g_ref[...] + be_ref[...]
        out3 = jnp.where(xn > 0, xn, jnp.exp(xn) - 1.0)
        e = jnp.exp(jk_ref[...])
        s = jnp.sum(e, axis=1, keepdims=True)
        z = (o1_ref[...] * e[:, 0:1] + o2_ref[...] * e[:, 1:2]
             + out3 * e[:, 2:3]) / s
        z_ref[...] = z
        zp_ref[...] = jnp.concatenate(
            [z, jnp.zeros((n, 128 - f), F32)], axis=1)

    return pl.pallas_call(
        body,
        out_shape=[jax.ShapeDtypeStruct((n, f), F32),
                   jax.ShapeDtypeStruct((n, 128), F32)],
        compiler_params=pltpu.CompilerParams(vmem_limit_bytes=100 * 2**20),
    )(aggr, t3, dinv, b, g, be, out1, out2, jk3)


def _tc_decoder(prod, tf2, tcf2, dw1, db1, dw2):
    bsz, f = prod.shape
    blk = 1024
    grid = (bsz // blk,)

    def body(p_ref, tf_ref, tcf_ref, w1_ref, b1_ref, w2_ref,
             lf_ref, lcf_ref):
        prod = p_ref[...]
        w1 = w1_ref[...]
        base = jnp.dot(prod, w1[:f, :], preferred_element_type=F32) + b1_ref[...]
        trow = w1[f:f + 1, :]
        w2 = w2_ref[...]
        hf = base + tf_ref[...] * trow
        hf = jnp.where(hf > 0, hf, jnp.exp(hf) - 1.0)
        lf_ref[...] = jnp.dot(hf, w2, preferred_element_type=F32)
        hc = base + tcf_ref[...] * trow
        hc = jnp.where(hc > 0, hc, jnp.exp(hc) - 1.0)
        lcf_ref[...] = jnp.dot(hc, w2, preferred_element_type=F32)

    h = dw1.shape[1]
    return pl.pallas_call(
        body,
        grid=grid,
        in_specs=[
            pl.BlockSpec((blk, f), lambda i: (i, 0)),
            pl.BlockSpec((blk, 1), lambda i: (i, 0)),
            pl.BlockSpec((blk, 1), lambda i: (i, 0)),
            pl.BlockSpec((f + 1, h), lambda i: (0, 0)),
            pl.BlockSpec((1, h), lambda i: (0, 0)),
            pl.BlockSpec((h, 1), lambda i: (0, 0)),
        ],
        out_specs=[
            pl.BlockSpec((blk, 1), lambda i: (i, 0)),
            pl.BlockSpec((blk, 1), lambda i: (i, 0)),
        ],
        out_shape=[jax.ShapeDtypeStruct((bsz, 1), F32),
                   jax.ShapeDtypeStruct((bsz, 1), F32)],
    )(prod, tf2, tcf2, dw1, db1, dw2)


def kernel(features, adj, edges, T_f_batch, T_cf_batch,
           W1, b1, W2, b2, W3, b3,
           g1, be1, g2, be2, g3, be3,
           jk_w, dW1, db1, dW2):
    n, d = features.shape
    e = adj.shape[1]
    bsz = edges.shape[0]

    adj = adj.astype(jnp.int32)
    edges = edges.astype(jnp.int32)

    ch = 128
    ch = 128
    ept = ((e // NW + 2 * ch - 1) // (2 * ch)) * (2 * ch)
    ppt = ept - e // NW
    spare = NACC - (n + 1) // 2
    pad_s = (jnp.arange(ppt, dtype=jnp.int32) * 37) % n
    pad_d = 2 * ((n + 1) // 2 + jnp.arange(ppt, dtype=jnp.int32) % spare)
    pad_s = jnp.broadcast_to(pad_s[None, :], (NW, ppt))
    pad_d = jnp.broadcast_to(pad_d[None, :], (NW, ppt))
    src1 = jnp.concatenate(
        [adj[0].reshape(NW, e // NW), pad_s], axis=1).reshape(-1)
    dst1 = jnp.concatenate(
        [adj[1].reshape(NW, e // NW), pad_d], axis=1).reshape(-1)
    dst2 = adj[1].reshape(e // 50, 50)
    npad = 10240

    degpart = _sc_degree(dst2, npad)
    degpart3 = degpart.reshape(NC, npad, 1)

    dinv, t3 = _tc_prep(features, W1, degpart3)

    b1r, g1r, be1r = b1.reshape(1, -1), g1.reshape(1, -1), be1.reshape(1, -1)
    b2r, g2r, be2r = b2.reshape(1, -1), g2.reshape(1, -1), be2.reshape(1, -1)
    b3r, g3r, be3r = b3.reshape(1, -1), g3.reshape(1, -1), be3.reshape(1, -1)

    agg1 = _sc_agg(t3.reshape(2 * n, 128), src1, dst1, ch)
    out1, t3 = _tc_combine(agg1.reshape(NC, 2 * NACC, 64), t3, dinv,
                           b1r, g1r, be1r, W2)
    agg2 = _sc_agg(t3.reshape(2 * n, 128), src1, dst1, ch)
    out2, t3 = _tc_combine(agg2.reshape(NC, 2 * NACC, 64), t3, dinv,
                           b2r, g2r, be2r, W3)
    agg3 = _sc_agg(t3.reshape(2 * n, 128), src1, dst1, ch)
    z, zpad = _tc_final(agg3.reshape(NC, 2 * NACC, 64), t3, dinv,
                        b3r, g3r, be3r, out1, out2, jk_w.reshape(1, 3))

    ech = 128
    ei2 = edges[:, 0].reshape(bsz // ech, ech)
    ej2 = edges[:, 1].reshape(bsz // ech, ech)
    prod = _sc_edge_hadamard(zpad, ei2, ej2, W3.shape[1])

    lf, lcf = _tc_decoder(prod,
                          T_f_batch.reshape(bsz, 1),
                          T_cf_batch.reshape(bsz, 1),
                          dW1, db1.reshape(1, -1), dW2)
    return (z, lf.reshape(bsz), lcf.reshape(bsz))


# speedup vs baseline: 1.1617x; 1.1617x over previous
import functools

import jax
import jax.numpy as jnp
from jax import lax
from jax.experimental import pallas as pl
from jax.experimental.pallas import tpu as pltpu
from jax.experimental.pallas import tpu_sc as plsc

F32 = jnp.float32
NC = 2
NS = 16
NW = NC * NS
NACC = 5120


def _mesh():
    return plsc.VectorSubcoreMesh(core_axis_name="c", subcore_axis_name="s")


def _col_groups(ch):
    offs = list(range(0, ch - 15, 16))
    if ch % 16:
        offs.append(ch - 16)
    return offs


def _sc_degree(dst2, n_pad):
    rt, ch = dst2.shape
    rows_per_tile = rt // NW
    seg = n_pad // NS

    @functools.partial(
        pl.kernel,
        out_type=jax.ShapeDtypeStruct((NC * n_pad,), F32),
        mesh=_mesh(),
        scratch_types=[
            pltpu.VMEM((rows_per_tile, ch), jnp.int32),
            pltpu.VMEM((ch,), F32),
            pltpu.VMEM((seg,), F32),
            pltpu.VMEM_SHARED((n_pad,), F32),
        ],
    )
    def deg_kernel(dst_hbm, out_hbm, idx_v, ones_v, zbuf_v, acc_sh):
        cid = lax.axis_index("c")
        sid = lax.axis_index("s")
        wid = sid * NC + cid

        def fill_zero(i, carry):
            zbuf_v[pl.ds(i * 16, 16)] = jnp.zeros((16,), F32)
            return carry

        lax.fori_loop(0, seg // 16, fill_zero, 0)
        for k in _col_groups(ch):
            ones_v[pl.ds(k, 16)] = jnp.ones((16,), F32)

        pltpu.sync_copy(zbuf_v, acc_sh.at[pl.ds(sid * seg, seg)])
        plsc.subcore_barrier()

        pltpu.sync_copy(dst_hbm.at[pl.ds(wid * rows_per_tile, rows_per_tile)],
                        idx_v)

        def body(j, carry):
            pltpu.sync_copy(ones_v, acc_sh.at[idx_v.at[j]], add=True)
            return carry

        lax.fori_loop(0, rows_per_tile, body, 0)

        plsc.subcore_barrier()
        pltpu.sync_copy(acc_sh.at[pl.ds(sid * seg, seg)],
                        out_hbm.at[pl.ds(cid * n_pad + sid * seg, seg)])

    return deg_kernel(dst2)


def _sc_agg(table2, src1, dst1, ch):
    e = src1.shape[0]
    ept = e // NW
    nchunk = ept // ch
    seg = NACC // NS
    cpy = 64

    @functools.partial(
        pl.kernel,
        out_type=jax.ShapeDtypeStruct((NC, NACC, 128), F32),
        mesh=_mesh(),
        scratch_types=[
            pltpu.VMEM((ept,), jnp.int32),
            pltpu.VMEM((ept,), jnp.int32),
            pltpu.VMEM((nchunk, ch), jnp.int32),
            pltpu.VMEM((ch, 128), F32),
            pltpu.VMEM((ch, 128), F32),
            pltpu.VMEM((cpy, 128), F32),
            pltpu.VMEM_SHARED((NACC, 128), F32),
            pltpu.SemaphoreType.DMA,
            pltpu.SemaphoreType.DMA,
        ],
    )
    def agg_kernel(t2_hbm, src_hbm, dst_hbm, out_hbm,
                   si_v, di_v, ai_v, rows0_v, rows1_v, zbuf_v, acc_sh,
                   sem0, sem1):
        cid = lax.axis_index("c")
        sid = lax.axis_index("s")
        wid = sid * NC + cid

        def zero_row(i, carry):
            for k in range(8):
                zbuf_v[i, pl.ds(k * 16, 16)] = jnp.zeros((16,), F32)
            return carry

        lax.fori_loop(0, cpy, zero_row, 0)
        for k in range(seg // cpy):
            pltpu.sync_copy(zbuf_v,
                            acc_sh.at[pl.ds(sid * seg + k * cpy, cpy)])
        plsc.subcore_barrier()

        pltpu.sync_copy(src_hbm.at[pl.ds(wid * ept, ept)], si_v)
        pltpu.sync_copy(dst_hbm.at[pl.ds(wid * ept, ept)], di_v)

        def xform_g(i, carry):
            s = si_v[pl.ds(i * 16, 16)]
            d = di_v[pl.ds(i * 16, 16)]
            si_v[pl.ds(i * 16, 16)] = s * 2 + (d & 1)
            return carry

        lax.fori_loop(0, ept // 16, xform_g, 0)

        groups = _col_groups(ch)

        def xform_a(r, carry):
            for co in groups:
                d = di_v[pl.ds(r * ch + co, 16)]
                ai_v[r, pl.ds(co, 16)] = d >> 1
            return carry

        lax.fori_loop(0, nchunk, xform_a, 0)

        def gstart(j, buf, sem):
            return pltpu.async_copy(
                t2_hbm.at[si_v.at[pl.ds(j * ch, ch)]], buf, sem)

        def gwait(j, buf, sem):
            pltpu.make_async_copy(
                t2_hbm.at[si_v.at[pl.ds(j * ch, ch)]], buf, sem).wait()

        gstart(0, rows0_v, sem0)

        def body2(jj, carry):
            j0 = jj * 2
            gstart(j0 + 1, rows1_v, sem1)
            gwait(j0, rows0_v, sem0)
            pltpu.sync_copy(rows0_v, acc_sh.at[ai_v.at[j0]], add=True)

            @pl.when(jj + 1 < nchunk // 2)
            def _():
                gstart(j0 + 2, rows0_v, sem0)

            gwait(j0 + 1, rows1_v, sem1)
            pltpu.sync_copy(rows1_v, acc_sh.at[ai_v.at[j0 + 1]], add=True)
            return carry

        lax.fori_loop(0, nchunk // 2, body2, 0)

        plsc.subcore_barrier()
        for k in range(seg // cpy):
            pltpu.sync_copy(acc_sh.at[pl.ds(sid * seg + k * cpy, cpy)],
                            out_hbm.at[cid, pl.ds(sid * seg + k * cpy, cpy)])

    return agg_kernel(table2, src1, dst1)


def _sc_edge_hadamard(zpad, ei2, ej2, f_out):
    n, f = zpad.shape
    rt, ch = ei2.shape
    rows_per_tile = rt // NW
    bsz = rt * ch

    @functools.partial(
        pl.kernel,
        out_type=jax.ShapeDtypeStruct((bsz, f_out), F32),
        mesh=_mesh(),
        scratch_types=[
            pltpu.VMEM((rows_per_tile, ch), jnp.int32),
            pltpu.VMEM((rows_per_tile, ch), jnp.int32),
            pltpu.VMEM((ch, f), F32),
            pltpu.VMEM((ch, f), F32),
            pltpu.VMEM((ch, f), F32),
            pltpu.VMEM((ch, f), F32),
            pltpu.VMEM((ch, f_out), F32),
            pltpu.VMEM((ch, f_out), F32),
            pltpu.SemaphoreType.DMA,
            pltpu.SemaphoreType.DMA,
        ],
    )
    def gat_kernel(z_hbm, ei_hbm, ej_hbm, prod_hbm,
                   ii_v, jj_v, bi0_v, bj0_v, bi1_v, bj1_v, pb0_v, pb1_v,
                   semA, semB):
        cid = lax.axis_index("c")
        sid = lax.axis_index("s")
        wid = sid * NC + cid
        base = wid * rows_per_tile * ch

        pltpu.sync_copy(ei_hbm.at[pl.ds(wid * rows_per_tile, rows_per_tile)],
                        ii_v)
        pltpu.sync_copy(ej_hbm.at[pl.ds(wid * rows_per_tile, rows_per_tile)],
                        jj_v)

        def gstart(j, bi, bj, sem):
            pltpu.async_copy(z_hbm.at[ii_v.at[j]], bi, sem)
            pltpu.async_copy(z_hbm.at[jj_v.at[j]], bj, sem)

        def gwait(j, bi, bj, sem):
            pltpu.make_async_copy(z_hbm.at[ii_v.at[j]], bi, sem).wait()
            pltpu.make_async_copy(z_hbm.at[jj_v.at[j]], bj, sem).wait()

        def consume(j, bi, bj, pb):
            def had(rr, c):
                for u in range(4):
                    r = rr * 4 + u
                    for k in range(f_out // 16):
                        a = bi[r, pl.ds(k * 16, 16)]
                        b = bj[r, pl.ds(k * 16, 16)]
                        pb[r, pl.ds(k * 16, 16)] = a * b
                return c

            lax.fori_loop(0, ch // 4, had, 0)
            pltpu.sync_copy(pb, prod_hbm.at[pl.ds(base + j * ch, ch)])

        gstart(0, bi0_v, bj0_v, semA)

        def body2(jj, carry):
            j0 = jj * 2
            gstart(j0 + 1, bi1_v, bj1_v, semB)
            gwait(j0, bi0_v, bj0_v, semA)
            consume(j0, bi0_v, bj0_v, pb0_v)

            @pl.when(jj + 1 < rows_per_tile // 2)
            def _():
                gstart(j0 + 2, bi0_v, bj0_v, semA)

            gwait(j0 + 1, bi1_v, bj1_v, semB)
            consume(j0 + 1, bi1_v, bj1_v, pb1_v)
            return carry

        lax.fori_loop(0, rows_per_tile // 2, body2, 0)

    return gat_kernel(zpad, ei2, ej2)


def _pack_pairs(hh, n, h):
    zpad = jnp.zeros((n, 128 - h), F32)
    row0 = jnp.concatenate([hh, zpad], axis=1)
    row1 = jnp.concatenate([zpad, hh], axis=1)
    return jnp.stack([row0, row1], axis=1)


def _tc_prep(features, w1, degpart3):
    n, d = features.shape
    h = w1.shape[1]
    npad = degpart3.shape[1]

    def mm_body(x_ref, w_ref, h_ref):
        h_ref[...] = jnp.dot(x_ref[...], w_ref[...],
                             preferred_element_type=F32)

    h1 = pl.pallas_call(
        mm_body,
        out_shape=jax.ShapeDtypeStruct((n, h), F32),
        compiler_params=pltpu.CompilerParams(vmem_limit_bytes=100 * 2**20),
    )(features, w1)

    def body(h_ref, deg_ref, dinv_ref, t3_ref):
        deg = deg_ref[...]
        degsum = deg[0] + deg[1] + 1.0
        dinv = lax.rsqrt(degsum[:n])
        dinv_ref[...] = dinv
        hh = h_ref[...] * dinv
        t3_ref[...] = _pack_pairs(hh, n, h)

    return pl.pallas_call(
        body,
        out_shape=[jax.ShapeDtypeStruct((n, 1), F32),
                   jax.ShapeDtypeStruct((n, 2, 128), F32)],
        compiler_params=pltpu.CompilerParams(vmem_limit_bytes=100 * 2**20),
    )(h1, degpart3)


def _tc_combine(aggr, t3, dinv, b, g, be, w_next):
    n = t3.shape[0]
    f = b.shape[1]
    h2 = w_next.shape[1]

    def body(agg_ref, t3_ref, dinv_ref, b_ref, g_ref, be_ref, wn_ref,
             out_ref, t3n_ref):
        agg = agg_ref[...]
        dinv = dinv_ref[...]
        hh = t3_ref[:, 0, :f]
        pre = (agg[0, :n] + agg[1, :n] + hh) * dinv + b_ref[...]
        m = jnp.mean(pre, axis=0, keepdims=True)
        xc = pre - m
        v = jnp.mean(xc * xc, axis=0, keepdims=True)
        xn = xc * lax.rsqrt(v + 1e-5) * g_ref[...] + be_ref[...]
        out = jnp.where(xn > 0, xn, jnp.exp(xn) - 1.0)
        out_ref[...] = out
        hhn = jnp.dot(out, wn_ref[...], preferred_element_type=F32) * dinv
        t3n_ref[...] = _pack_pairs(hhn, n, h2)

    return pl.pallas_call(
        body,
        out_shape=[jax.ShapeDtypeStruct((n, f), F32),
                   jax.ShapeDtypeStruct((n, 2, 128), F32)],
        compiler_params=pltpu.CompilerParams(vmem_limit_bytes=100 * 2**20),
    )(aggr, t3, dinv, b, g, be, w_next)


def _tc_final(aggr, t3, dinv, b, g, be, out1, out2, jk3):
    n = t3.shape[0]
    f = b.shape[1]

    def body(agg_ref, t3_ref, dinv_ref, b_ref, g_ref, be_ref,
             o1_ref, o2_ref, jk_ref, z_ref, zp_ref):
        agg = agg_ref[...]
        hh = t3_ref[:, 0, :f]
        pre = (agg[0, :n] + agg[1, :n] + hh) * dinv_ref[...] + b_ref[...]
        m = jnp.mean(pre, axis=0, keepdims=True)
        xc = pre - m
        v = jnp.mean(xc * xc, axis=0, keepdims=True)
        xn = xc * lax.rsqrt(v + 1e-5) * g_ref[...] + be_ref[...]
        out3 = jnp.where(xn > 0, xn, jnp.exp(xn) - 1.0)
        e = jnp.exp(jk_ref[...])
        s = jnp.sum(e, axis=1, keepdims=True)
        z = (o1_ref[...] * e[:, 0:1] + o2_ref[...] * e[:, 1:2]
             + out3 * e[:, 2:3]) / s
        z_ref[...] = z
        zp_ref[...] = jnp.concatenate(
            [z, jnp.zeros((n, 128 - f), F32)], axis=1)

    return pl.pallas_call(
        body,
        out_shape=[jax.ShapeDtypeStruct((n, f), F32),
                   jax.ShapeDtypeStruct((n, 128), F32)],
        compiler_params=pltpu.CompilerParams(vmem_limit_bytes=100 * 2**20),
    )(aggr, t3, dinv, b, g, be, out1, out2, jk3)


def _tc_decoder(prod, tf2, tcf2, dw1, db1, dw2):
    bsz, f = prod.shape
    blk = 1024
    grid = (bsz // blk,)

    def body(p_ref, tf_ref, tcf_ref, w1_ref, b1_ref, w2_ref,
             lf_ref, lcf_ref):
        prod = p_ref[...]
        w1 = w1_ref[...]
        base = jnp.dot(prod, w1[:f, :], preferred_element_type=F32) + b1_ref[...]
        trow = w1[f:f + 1, :]
        w2 = w2_ref[...]
        hf = base + tf_ref[...] * trow
        hf = jnp.where(hf > 0, hf, jnp.exp(hf) - 1.0)
        lf_ref[...] = jnp.dot(hf, w2, preferred_element_type=F32)
        hc = base + tcf_ref[...] * trow
        hc = jnp.where(hc > 0, hc, jnp.exp(hc) - 1.0)
        lcf_ref[...] = jnp.dot(hc, w2, preferred_element_type=F32)

    h = dw1.shape[1]
    return pl.pallas_call(
        body,
        grid=grid,
        in_specs=[
            pl.BlockSpec((blk, f), lambda i: (i, 0)),
            pl.BlockSpec((blk, 1), lambda i: (i, 0)),
            pl.BlockSpec((blk, 1), lambda i: (i, 0)),
            pl.BlockSpec((f + 1, h), lambda i: (0, 0)),
            pl.BlockSpec((1, h), lambda i: (0, 0)),
            pl.BlockSpec((h, 1), lambda i: (0, 0)),
        ],
        out_specs=[
            pl.BlockSpec((blk, 1), lambda i: (i, 0)),
            pl.BlockSpec((blk, 1), lambda i: (i, 0)),
        ],
        out_shape=[jax.ShapeDtypeStruct((bsz, 1), F32),
                   jax.ShapeDtypeStruct((bsz, 1), F32)],
    )(prod, tf2, tcf2, dw1, db1, dw2)


def kernel(features, adj, edges, T_f_batch, T_cf_batch,
           W1, b1, W2, b2, W3, b3,
           g1, be1, g2, be2, g3, be3,
           jk_w, dW1, db1, dW2):
    n, d = features.shape
    e = adj.shape[1]
    bsz = edges.shape[0]

    adj = adj.astype(jnp.int32)
    edges = edges.astype(jnp.int32)

    ch = 128
    ch = 128
    ept = ((e // NW + 2 * ch - 1) // (2 * ch)) * (2 * ch)
    ppt = ept - e // NW
    spare = NACC - (n + 1) // 2
    pad_s = (jnp.arange(ppt, dtype=jnp.int32) * 37) % n
    pad_d = 2 * ((n + 1) // 2 + jnp.arange(ppt, dtype=jnp.int32) % spare)
    pad_s = jnp.broadcast_to(pad_s[None, :], (NW, ppt))
    pad_d = jnp.broadcast_to(pad_d[None, :], (NW, ppt))
    src1 = jnp.concatenate(
        [adj[0].reshape(NW, e // NW), pad_s], axis=1).reshape(-1)
    dst1 = jnp.concatenate(
        [adj[1].reshape(NW, e // NW), pad_d], axis=1).reshape(-1)
    dst2 = adj[1].reshape(e // 50, 50)
    npad = 10240

    degpart = _sc_degree(dst2, npad)
    degpart3 = degpart.reshape(NC, npad, 1)

    dinv, t3 = _tc_prep(features, W1, degpart3)

    b1r, g1r, be1r = b1.reshape(1, -1), g1.reshape(1, -1), be1.reshape(1, -1)
    b2r, g2r, be2r = b2.reshape(1, -1), g2.reshape(1, -1), be2.reshape(1, -1)
    b3r, g3r, be3r = b3.reshape(1, -1), g3.reshape(1, -1), be3.reshape(1, -1)

    agg1 = _sc_agg(t3.reshape(2 * n, 128), src1, dst1, ch)
    out1, t3 = _tc_combine(agg1.reshape(NC, 2 * NACC, 64), t3, dinv,
                           b1r, g1r, be1r, W2)
    agg2 = _sc_agg(t3.reshape(2 * n, 128), src1, dst1, ch)
    out2, t3 = _tc_combine(agg2.reshape(NC, 2 * NACC, 64), t3, dinv,
                           b2r, g2r, be2r, W3)
    agg3 = _sc_agg(t3.reshape(2 * n, 128), src1, dst1, ch)
    z, zpad = _tc_final(agg3.reshape(NC, 2 * NACC, 64), t3, dinv,
                        b3r, g3r, be3r, out1, out2, jk_w.reshape(1, 3))

    ech = 128
    ei2 = edges[:, 0].reshape(bsz // ech, ech)
    ej2 = edges[:, 1].reshape(bsz // ech, ech)
    prod = _sc_edge_hadamard(zpad, ei2, ej2, W3.shape[1])

    lf, lcf = _tc_decoder(prod,
                          T_f_batch.reshape(bsz, 1),
                          T_cf_batch.reshape(bsz, 1),
                          dW1, db1.reshape(1, -1), dW2)
    return (z, lf.reshape(bsz), lcf.reshape(bsz))


# speedup vs baseline: 1.1902x; 1.0246x over previous
import functools

import jax
import jax.numpy as jnp
from jax import lax
from jax.experimental import pallas as pl
from jax.experimental.pallas import tpu as pltpu
from jax.experimental.pallas import tpu_sc as plsc

F32 = jnp.float32
NC = 2
NS = 16
NW = NC * NS
NACC = 5120


def _mesh():
    return plsc.VectorSubcoreMesh(core_axis_name="c", subcore_axis_name="s")


def _col_groups(ch):
    offs = list(range(0, ch - 15, 16))
    if ch % 16:
        offs.append(ch - 16)
    return offs


def _sc_degree(dst2, n_pad):
    rt, ch = dst2.shape
    rows_per_tile = rt // NW
    seg = n_pad // NS

    @functools.partial(
        pl.kernel,
        out_type=jax.ShapeDtypeStruct((NC * n_pad,), F32),
        mesh=_mesh(),
        scratch_types=[
            pltpu.VMEM((rows_per_tile, ch), jnp.int32),
            pltpu.VMEM((ch,), F32),
            pltpu.VMEM((seg,), F32),
            pltpu.VMEM_SHARED((n_pad,), F32),
        ],
    )
    def deg_kernel(dst_hbm, out_hbm, idx_v, ones_v, zbuf_v, acc_sh):
        cid = lax.axis_index("c")
        sid = lax.axis_index("s")
        wid = sid * NC + cid

        def fill_zero(i, carry):
            zbuf_v[pl.ds(i * 16, 16)] = jnp.zeros((16,), F32)
            return carry

        lax.fori_loop(0, seg // 16, fill_zero, 0)
        for k in _col_groups(ch):
            ones_v[pl.ds(k, 16)] = jnp.ones((16,), F32)

        pltpu.sync_copy(zbuf_v, acc_sh.at[pl.ds(sid * seg, seg)])
        plsc.subcore_barrier()

        pltpu.sync_copy(dst_hbm.at[pl.ds(wid * rows_per_tile, rows_per_tile)],
                        idx_v)

        def body(j, carry):
            pltpu.sync_copy(ones_v, acc_sh.at[idx_v.at[j]], add=True)
            return carry

        lax.fori_loop(0, rows_per_tile, body, 0)

        plsc.subcore_barrier()
        pltpu.sync_copy(acc_sh.at[pl.ds(sid * seg, seg)],
                        out_hbm.at[pl.ds(cid * n_pad + sid * seg, seg)])

    return deg_kernel(dst2)


def _sc_agg(table2, src1, dst1, ch):
    nrows = table2.shape[0] // 2
    e = src1.shape[0]
    ept = e // NW
    nchunk = ept // ch
    seg = NACC // NS
    cpy = 64

    @functools.partial(
        pl.kernel,
        out_type=jax.ShapeDtypeStruct((NC, NACC, 128), F32),
        mesh=_mesh(),
        scratch_types=[
            pltpu.VMEM((ept,), jnp.int32),
            pltpu.VMEM((ept,), jnp.int32),
            pltpu.VMEM((nchunk, ch), jnp.int32),
            pltpu.VMEM((ch, 128), F32),
            pltpu.VMEM((ch, 128), F32),
            pltpu.VMEM((cpy, 128), F32),
            pltpu.VMEM_SHARED((NACC, 128), F32),
            pltpu.SemaphoreType.DMA,
            pltpu.SemaphoreType.DMA,
        ],
    )
    def agg_kernel(t2_hbm, src_hbm, dst_hbm, out_hbm,
                   si_v, di_v, ai_v, rows0_v, rows1_v, zbuf_v, acc_sh,
                   sem0, sem1):
        cid = lax.axis_index("c")
        sid = lax.axis_index("s")
        wid = sid * NC + cid

        def zero_row(i, carry):
            for k in range(8):
                zbuf_v[i, pl.ds(k * 16, 16)] = jnp.zeros((16,), F32)
            return carry

        lax.fori_loop(0, cpy, zero_row, 0)
        for k in range(seg // cpy):
            pltpu.sync_copy(zbuf_v,
                            acc_sh.at[pl.ds(sid * seg + k * cpy, cpy)])
        plsc.subcore_barrier()

        pltpu.sync_copy(src_hbm.at[pl.ds(wid * ept, ept)], si_v)
        pltpu.sync_copy(dst_hbm.at[pl.ds(wid * ept, ept)], di_v)

        def xform_g(i, carry):
            s = si_v[pl.ds(i * 16, 16)]
            d = di_v[pl.ds(i * 16, 16)]
            si_v[pl.ds(i * 16, 16)] = s + (d & 1) * nrows
            return carry

        lax.fori_loop(0, ept // 16, xform_g, 0)

        groups = _col_groups(ch)

        def xform_a(r, carry):
            for co in groups:
                d = di_v[pl.ds(r * ch + co, 16)]
                ai_v[r, pl.ds(co, 16)] = d >> 1
            return carry

        lax.fori_loop(0, nchunk, xform_a, 0)

        def gstart(j, buf, sem):
            return pltpu.async_copy(
                t2_hbm.at[si_v.at[pl.ds(j * ch, ch)]], buf, sem)

        def gwait(j, buf, sem):
            pltpu.make_async_copy(
                t2_hbm.at[si_v.at[pl.ds(j * ch, ch)]], buf, sem).wait()

        gstart(0, rows0_v, sem0)

        def body2(jj, carry):
            j0 = jj * 2
            gstart(j0 + 1, rows1_v, sem1)
            gwait(j0, rows0_v, sem0)
            pltpu.sync_copy(rows0_v, acc_sh.at[ai_v.at[j0]], add=True)

            @pl.when(jj + 1 < nchunk // 2)
            def _():
                gstart(j0 + 2, rows0_v, sem0)

            gwait(j0 + 1, rows1_v, sem1)
            pltpu.sync_copy(rows1_v, acc_sh.at[ai_v.at[j0 + 1]], add=True)
            return carry

        lax.fori_loop(0, nchunk // 2, body2, 0)

        plsc.subcore_barrier()
        for k in range(seg // cpy):
            pltpu.sync_copy(acc_sh.at[pl.ds(sid * seg + k * cpy, cpy)],
                            out_hbm.at[cid, pl.ds(sid * seg + k * cpy, cpy)])

    return agg_kernel(table2, src1, dst1)


def _sc_edge_hadamard(zpad, ei2, ej2, f_out):
    n, f = zpad.shape
    rt, ch = ei2.shape
    rows_per_tile = rt // NW
    bsz = rt * ch

    @functools.partial(
        pl.kernel,
        out_type=jax.ShapeDtypeStruct((bsz, f_out), F32),
        mesh=_mesh(),
        scratch_types=[
            pltpu.VMEM((rows_per_tile, ch), jnp.int32),
            pltpu.VMEM((rows_per_tile, ch), jnp.int32),
            pltpu.VMEM((ch, f), F32),
            pltpu.VMEM((ch, f), F32),
            pltpu.VMEM((ch, f), F32),
            pltpu.VMEM((ch, f), F32),
            pltpu.VMEM((ch, f_out), F32),
            pltpu.VMEM((ch, f_out), F32),
            pltpu.SemaphoreType.DMA,
            pltpu.SemaphoreType.DMA,
        ],
    )
    def gat_kernel(z_hbm, ei_hbm, ej_hbm, prod_hbm,
                   ii_v, jj_v, bi0_v, bj0_v, bi1_v, bj1_v, pb0_v, pb1_v,
                   semA, semB):
        cid = lax.axis_index("c")
        sid = lax.axis_index("s")
        wid = sid * NC + cid
        base = wid * rows_per_tile * ch

        pltpu.sync_copy(ei_hbm.at[pl.ds(wid * rows_per_tile, rows_per_tile)],
                        ii_v)
        pltpu.sync_copy(ej_hbm.at[pl.ds(wid * rows_per_tile, rows_per_tile)],
                        jj_v)

        def gstart(j, bi, bj, sem):
            pltpu.async_copy(z_hbm.at[ii_v.at[j]], bi, sem)
            pltpu.async_copy(z_hbm.at[jj_v.at[j]], bj, sem)

        def gwait(j, bi, bj, sem):
            pltpu.make_async_copy(z_hbm.at[ii_v.at[j]], bi, sem).wait()
            pltpu.make_async_copy(z_hbm.at[jj_v.at[j]], bj, sem).wait()

        def consume(j, bi, bj, pb):
            def had(rr, c):
                for u in range(4):
                    r = rr * 4 + u
                    for k in range(f_out // 16):
                        a = bi[r, pl.ds(k * 16, 16)]
                        b = bj[r, pl.ds(k * 16, 16)]
                        pb[r, pl.ds(k * 16, 16)] = a * b
                return c

            lax.fori_loop(0, ch // 4, had, 0)
            pltpu.sync_copy(pb, prod_hbm.at[pl.ds(base + j * ch, ch)])

        gstart(0, bi0_v, bj0_v, semA)

        def body2(jj, carry):
            j0 = jj * 2
            gstart(j0 + 1, bi1_v, bj1_v, semB)
            gwait(j0, bi0_v, bj0_v, semA)
            consume(j0, bi0_v, bj0_v, pb0_v)

            @pl.when(jj + 1 < rows_per_tile // 2)
            def _():
                gstart(j0 + 2, bi0_v, bj0_v, semA)

            gwait(j0 + 1, bi1_v, bj1_v, semB)
            consume(j0 + 1, bi1_v, bj1_v, pb1_v)
            return carry

        lax.fori_loop(0, rows_per_tile // 2, body2, 0)

    return gat_kernel(zpad, ei2, ej2)


def _pack_pairs(hh, n, h):
    zpad = jnp.zeros((n, 128 - h), F32)
    row0 = jnp.concatenate([hh, zpad], axis=1)
    row1 = jnp.concatenate([zpad, hh], axis=1)
    return jnp.stack([row0, row1], axis=0)


def _tc_prep(features, w1, degpart3):
    n, d = features.shape
    h = w1.shape[1]
    npad = degpart3.shape[1]

    def mm_body(x_ref, w_ref, h_ref):
        h_ref[...] = jnp.dot(x_ref[...], w_ref[...],
                             preferred_element_type=F32)

    h1 = pl.pallas_call(
        mm_body,
        out_shape=jax.ShapeDtypeStruct((n, h), F32),
        compiler_params=pltpu.CompilerParams(vmem_limit_bytes=100 * 2**20),
    )(features, w1)

    def body(h_ref, deg_ref, dinv_ref, t3_ref):
        deg = deg_ref[...]
        degsum = deg[0] + deg[1] + 1.0
        dinv = lax.rsqrt(degsum[:n])
        dinv_ref[...] = dinv
        hh = h_ref[...] * dinv
        t3_ref[...] = _pack_pairs(hh, n, h)

    return pl.pallas_call(
        body,
        out_shape=[jax.ShapeDtypeStruct((n, 1), F32),
                   jax.ShapeDtypeStruct((2, n, 128), F32)],
        compiler_params=pltpu.CompilerParams(vmem_limit_bytes=100 * 2**20),
    )(h1, degpart3)


def _tc_combine(aggr, t3, dinv, b, g, be, w_next):
    n = t3.shape[1]
    f = b.shape[1]
    h2 = w_next.shape[1]

    def body(agg_ref, t3_ref, dinv_ref, b_ref, g_ref, be_ref, wn_ref,
             out_ref, t3n_ref):
        agg = agg_ref[...]
        dinv = dinv_ref[...]
        hh = t3_ref[0, :, :f]
        pre = (agg[0, :n] + agg[1, :n] + hh) * dinv + b_ref[...]
        m = jnp.mean(pre, axis=0, keepdims=True)
        xc = pre - m
        v = jnp.mean(xc * xc, axis=0, keepdims=True)
        xn = xc * lax.rsqrt(v + 1e-5) * g_ref[...] + be_ref[...]
        out = jnp.where(xn > 0, xn, jnp.exp(xn) - 1.0)
        out_ref[...] = out
        hhn = jnp.dot(out, wn_ref[...], preferred_element_type=F32) * dinv
        t3n_ref[...] = _pack_pairs(hhn, n, h2)

    return pl.pallas_call(
        body,
        out_shape=[jax.ShapeDtypeStruct((n, f), F32),
                   jax.ShapeDtypeStruct((2, n, 128), F32)],
        compiler_params=pltpu.CompilerParams(vmem_limit_bytes=100 * 2**20),
    )(aggr, t3, dinv, b, g, be, w_next)


def _tc_final(aggr, t3, dinv, b, g, be, out1, out2, jk3):
    n = t3.shape[1]
    f = b.shape[1]

    def body(agg_ref, t3_ref, dinv_ref, b_ref, g_ref, be_ref,
             o1_ref, o2_ref, jk_ref, z_ref, zp_ref):
        agg = agg_ref[...]
        hh = t3_ref[0, :, :f]
        pre = (agg[0, :n] + agg[1, :n] + hh) * dinv_ref[...] + b_ref[...]
        m = jnp.mean(pre, axis=0, keepdims=True)
        xc = pre - m
        v = jnp.mean(xc * xc, axis=0, keepdims=True)
        xn = xc * lax.rsqrt(v + 1e-5) * g_ref[...] + be_ref[...]
        out3 = jnp.where(xn > 0, xn, jnp.exp(xn) - 1.0)
        e = jnp.exp(jk_ref[...])
        s = jnp.sum(e, axis=1, keepdims=True)
        z = (o1_ref[...] * e[:, 0:1] + o2_ref[...] * e[:, 1:2]
             + out3 * e[:, 2:3]) / s
        z_ref[...] = z
        zp_ref[...] = jnp.concatenate(
            [z, jnp.zeros((n, 128 - f), F32)], axis=1)

    return pl.pallas_call(
        body,
        out_shape=[jax.ShapeDtypeStruct((n, f), F32),
                   jax.ShapeDtypeStruct((n, 128), F32)],
        compiler_params=pltpu.CompilerParams(vmem_limit_bytes=100 * 2**20),
    )(aggr, t3, dinv, b, g, be, out1, out2, jk3)


def _tc_decoder(prod, tf2, tcf2, dw1, db1, dw2):
    bsz, f = prod.shape
    blk = 1024
    grid = (bsz // blk,)

    def body(p_ref, tf_ref, tcf_ref, w1_ref, b1_ref, w2_ref,
             lf_ref, lcf_ref):
        prod = p_ref[...]
        w1 = w1_ref[...]
        base = jnp.dot(prod, w1[:f, :], preferred_element_type=F32) + b1_ref[...]
        trow = w1[f:f + 1, :]
        w2 = w2_ref[...]
        hf = base + tf_ref[...] * trow
        hf = jnp.where(hf > 0, hf, jnp.exp(hf) - 1.0)
        lf_ref[...] = jnp.dot(hf, w2, preferred_element_type=F32)
        hc = base + tcf_ref[...] * trow
        hc = jnp.where(hc > 0, hc, jnp.exp(hc) - 1.0)
        lcf_ref[...] = jnp.dot(hc, w2, preferred_element_type=F32)

    h = dw1.shape[1]
    return pl.pallas_call(
        body,
        grid=grid,
        in_specs=[
            pl.BlockSpec((blk, f), lambda i: (i, 0)),
            pl.BlockSpec((blk, 1), lambda i: (i, 0)),
            pl.BlockSpec((blk, 1), lambda i: (i, 0)),
            pl.BlockSpec((f + 1, h), lambda i: (0, 0)),
            pl.BlockSpec((1, h), lambda i: (0, 0)),
            pl.BlockSpec((h, 1), lambda i: (0, 0)),
        ],
        out_specs=[
            pl.BlockSpec((blk, 1), lambda i: (i, 0)),
            pl.BlockSpec((blk, 1), lambda i: (i, 0)),
        ],
        out_shape=[jax.ShapeDtypeStruct((bsz, 1), F32),
                   jax.ShapeDtypeStruct((bsz, 1), F32)],
    )(prod, tf2, tcf2, dw1, db1, dw2)


def kernel(features, adj, edges, T_f_batch, T_cf_batch,
           W1, b1, W2, b2, W3, b3,
           g1, be1, g2, be2, g3, be3,
           jk_w, dW1, db1, dW2):
    n, d = features.shape
    e = adj.shape[1]
    bsz = edges.shape[0]

    adj = adj.astype(jnp.int32)
    edges = edges.astype(jnp.int32)

    ch = 128
    ch = 128
    ept = ((e // NW + 2 * ch - 1) // (2 * ch)) * (2 * ch)
    ppt = ept - e // NW
    spare = NACC - (n + 1) // 2
    pad_s = (jnp.arange(ppt, dtype=jnp.int32) * 37) % n
    pad_d = 2 * ((n + 1) // 2 + jnp.arange(ppt, dtype=jnp.int32) % spare)
    pad_s = jnp.broadcast_to(pad_s[None, :], (NW, ppt))
    pad_d = jnp.broadcast_to(pad_d[None, :], (NW, ppt))
    src1 = jnp.concatenate(
        [adj[0].reshape(NW, e // NW), pad_s], axis=1).reshape(-1)
    dst1 = jnp.concatenate(
        [adj[1].reshape(NW, e // NW), pad_d], axis=1).reshape(-1)
    dst2 = adj[1].reshape(e // 50, 50)
    npad = 10240

    degpart = _sc_degree(dst2, npad)
    degpart3 = degpart.reshape(NC, npad, 1)

    dinv, t3 = _tc_prep(features, W1, degpart3)

    b1r, g1r, be1r = b1.reshape(1, -1), g1.reshape(1, -1), be1.reshape(1, -1)
    b2r, g2r, be2r = b2.reshape(1, -1), g2.reshape(1, -1), be2.reshape(1, -1)
    b3r, g3r, be3r = b3.reshape(1, -1), g3.reshape(1, -1), be3.reshape(1, -1)

    agg1 = _sc_agg(t3.reshape(2 * n, 128), src1, dst1, ch)
    out1, t3 = _tc_combine(agg1.reshape(NC, 2 * NACC, 64), t3, dinv,
                           b1r, g1r, be1r, W2)
    agg2 = _sc_agg(t3.reshape(2 * n, 128), src1, dst1, ch)
    out2, t3 = _tc_combine(agg2.reshape(NC, 2 * NACC, 64), t3, dinv,
                           b2r, g2r, be2r, W3)
    agg3 = _sc_agg(t3.reshape(2 * n, 128), src1, dst1, ch)
    z, zpad = _tc_final(agg3.reshape(NC, 2 * NACC, 64), t3, dinv,
                        b3r, g3r, be3r, out1, out2, jk_w.reshape(1, 3))

    ech = 128
    ei2 = edges[:, 0].reshape(bsz // ech, ech)
    ej2 = edges[:, 1].reshape(bsz // ech, ech)
    prod = _sc_edge_hadamard(zpad, ei2, ej2, W3.shape[1])

    lf, lcf = _tc_decoder(prod,
                          T_f_batch.reshape(bsz, 1),
                          T_cf_batch.reshape(bsz, 1),
                          dW1, db1.reshape(1, -1), dW2)
    return (z, lf.reshape(bsz), lcf.reshape(bsz))
